# baseline (speedup 1.0000x reference)
import functools

import jax
import jax.numpy as jnp
from jax import lax
from jax.experimental import pallas as pl
from jax.experimental.pallas import tpu as pltpu
from jax.experimental.pallas import tpu_sc as plsc

_RROWS = 1184
_NSLOT = 4


@functools.lru_cache(maxsize=None)
def _make_gather(num_indices: int, vocab: int, dim: int, dtype):
    info = plsc.get_sparse_core_info()
    nc, ns = info.num_cores, info.num_subcores
    per_w = num_indices // ns
    half = -(-vocab // (nc * 8)) * 8
    npass = -(-half // _RROWS)
    npass += npass % 2
    wrows = _RROWS + (vocab - _RROWS) % 8
    jbits = (per_w - 1).bit_length()
    assert per_w % 16 == 0 and num_indices % ns == 0
    assert (wrows - 1).bit_length() + jbits < 31
    assert vocab >= wrows and _RROWS % 8 == 0

    mesh = plsc.VectorSubcoreMesh(
        core_axis_name="core", subcore_axis_name="subcore"
    )

    @functools.partial(
        pl.kernel,
        out_type=jax.ShapeDtypeStruct((num_indices, dim), dtype),
        mesh=mesh,
        compiler_params=pltpu.CompilerParams(needs_layout_passes=False, use_tc_tiling_on_sc=False),
        scratch_types=[
            pltpu.VMEM((per_w,), jnp.int32),
            pltpu.VMEM((per_w + 128,), jnp.int32),
            pltpu.VMEM_SHARED((2, wrows, dim), dtype),
        ]
        + [pltpu.SemaphoreType.DMA for _ in range(6)],
    )
    def gather_kernel(table_hbm, ids_hbm, out_hbm, idx_v, sel_v, spm,
                      *rest):
        sem_r = rest[:2]
        ss = rest[2:6]

        sid = lax.axis_index("subcore")
        cid = lax.axis_index("core")
        base = sid * per_w
        cb = cid * half
        lane = jax.lax.iota(jnp.int32, 16)

        def range_start(p):
            return pl.multiple_of(
                jnp.minimum(cb + p * _RROWS, vocab - wrows), 8)

        def issue_range_load(p, slot):
            pltpu.async_copy(
                table_hbm.at[pl.ds(range_start(p), wrows)],
                spm.at[slot], sem_r[slot])

        def wait_range_load(slot):
            pltpu.make_async_copy(
                table_hbm.at[pl.ds(0, wrows)], spm.at[slot],
                sem_r[slot]).wait()

        @pl.when(sid == 0)
        def _():
            issue_range_load(0, 0)
        pltpu.sync_copy(ids_hbm.at[pl.ds(base, per_w)], idx_v)

        def do_pass(p, slot):
            @pl.when(sid == 0)
            def _():
                wait_range_load(slot)
            plsc.subcore_barrier()

            @pl.when(jnp.logical_and(sid == 0, p + 1 < npass))
            def _():
                issue_range_load(p + 1, 1 - slot)

            lo = cb + p * _RROWS
            hi = jnp.minimum(lo + _RROWS, cb + half)
            start = range_start(p)

            def scan_step(k, cursor):
                v = idx_v[pl.ds(k * 16, 16)]
                m = (v >= lo) & (v < hi)
                packed = ((v - start) << jbits) | (lane + k * 16)
                plsc.store_compressed(
                    sel_v.at[pl.ds(cursor, 16)], packed, mask=m)
                return cursor + plsc.all_reduce_population_count(m)[0]

            cnt = lax.fori_loop(0, per_w // 16, scan_step, 0,
                                unroll=False)

            @pl.when(cnt > 0)
            def _():
                e0 = jnp.full((16,), sel_v[pl.ds(0, 16)][0],
                              dtype=jnp.int32)
                rd = (cnt // 16) * 16
                cur = sel_v[pl.ds(rd, 16)]
                sel_v[pl.ds(rd, 16)] = jnp.where(lane < cnt - rd, cur, e0)
                for t in range(1, _NSLOT):
                    sel_v[pl.ds(rd + 16 * t, 16)] = e0

                nch = (cnt + 15) // 16
                nquad = (nch + 3) // 4

                def group_copy(g, sem):
                    pv = sel_v[pl.ds(g * 16, 16)]
                    for i in range(16):
                        pvi = pv[i]
                        pltpu.async_copy(
                            spm.at[slot].at[pl.ds(pvi >> jbits, 1)],
                            out_hbm.at[pl.ds(
                                (pvi & ((1 << jbits) - 1)) + base, 1)],
                            sem)

                def wait_group(sem):
                    pltpu.make_async_copy(
                        spm.at[slot].at[pl.ds(0, 16)],
                        out_hbm.at[pl.ds(0, 16)], sem).wait()

                def quad(it, carry):
                    for b in range(4):
                        @pl.when(it > 0)
                        def _():
                            wait_group(ss[b])
                        group_copy(it * 4 + b, ss[b])
                    return carry

                lax.fori_loop(0, nquad, quad, 0, unroll=False)
                for b in range(4):
                    wait_group(ss[b])

        @pl.loop(0, npass, step=2)
        def _(p2):
            do_pass(p2, 0)
            do_pass(p2 + 1, 1)

    return gather_kernel


def kernel(token_ids, embedding_table):
    b, s = token_ids.shape
    v, d = embedding_table.shape
    n = b * s
    ids = token_ids.reshape(n).astype(jnp.int32)
    out = _make_gather(n, v, d, embedding_table.dtype)(
        embedding_table, ids)
    return out.reshape(b, s, d)


# speedup vs baseline: 3.0290x; 3.0290x over previous
import functools

import jax
import jax.numpy as jnp
from jax import lax
from jax.experimental import pallas as pl
from jax.experimental.pallas import tpu as pltpu
from jax.experimental.pallas import tpu_sc as plsc

_CHUNK = 16
_NBUF = 8


@functools.lru_cache(maxsize=None)
def _make_gather(num_indices: int, dim: int, dtype):
    info = plsc.get_sparse_core_info()
    nw = info.num_cores * info.num_subcores
    per_w = num_indices // nw
    nchunk = per_w // _CHUNK
    look = _NBUF // 2
    assert num_indices % (nw * _CHUNK) == 0
    assert nchunk % _NBUF == 0 and nchunk >= 2 * _NBUF

    mesh = plsc.VectorSubcoreMesh(
        core_axis_name="core", subcore_axis_name="subcore"
    )

    @functools.partial(
        pl.kernel,
        out_type=jax.ShapeDtypeStruct((num_indices, dim), dtype),
        mesh=mesh,
        scratch_types=[
            pltpu.VMEM((per_w,), jnp.int32),
        ]
        + [pltpu.VMEM((_CHUNK, dim), dtype) for _ in range(_NBUF)]
        + [pltpu.SemaphoreType.DMA for _ in range(2 * _NBUF)],
    )
    def gather_kernel(table_hbm, ids_hbm, out_hbm, idx_v, *rest):
        bufs = rest[:_NBUF]
        sg = rest[_NBUF:2 * _NBUF]
        ss = rest[2 * _NBUF:3 * _NBUF]

        wid = (lax.axis_index("subcore") * info.num_cores
               + lax.axis_index("core"))
        base = wid * per_w
        pltpu.sync_copy(ids_hbm.at[pl.ds(base, per_w)], idx_v)

        def issue_gather(cc, b):
            pltpu.async_copy(
                table_hbm.at[idx_v.at[pl.ds(cc * _CHUNK, _CHUNK)]],
                bufs[b], sg[b])

        def wait_gather(b):
            pltpu.make_async_copy(
                table_hbm.at[pl.ds(0, _CHUNK)], bufs[b], sg[b]).wait()

        def issue_store(cc, b):
            pltpu.async_copy(
                bufs[b], out_hbm.at[pl.ds(base + cc * _CHUNK, _CHUNK)],
                ss[b])

        def wait_store(b):
            pltpu.make_async_copy(
                bufs[b], out_hbm.at[pl.ds(0, _CHUNK)], ss[b]).wait()

        def visit(cc, b, prefetch=True, free=True):
            bn = (b + look) % _NBUF
            if free:
                wait_store(bn)
            if prefetch:
                issue_gather(cc + look, bn)
            wait_gather(b)
            issue_store(cc, b)

        for b in range(look):
            issue_gather(b, b)
        for b in range(_NBUF):
            visit(b, b, free=(b >= look))

        @pl.loop(_NBUF, nchunk - _NBUF, step=_NBUF)
        def _(c):
            for b in range(_NBUF):
                visit(c + b, b)

        for b in range(_NBUF):
            visit(nchunk - _NBUF + b, b, prefetch=(b < look),
                  free=(b < look))
        for b in range(_NBUF):
            wait_store(b)

    return gather_kernel


def kernel(token_ids, embedding_table):
    b, s = token_ids.shape
    _, d = embedding_table.shape
    n = b * s
    ids = token_ids.reshape(n).astype(jnp.int32)
    out = _make_gather(n, d, embedding_table.dtype)(embedding_table, ids)
    return out.reshape(b, s, d)
